# Initial kernel scaffold; baseline (speedup 1.0000x reference)
import jax
import jax.numpy as jnp
from jax.experimental import pallas as pl


def kernel(tgt, feat0, feat1, feat2, feat3, Wq, bq, Wkv0, bkv0, Wkv1, bkv1, Wkv2, bkv2, Wkv3, bkv3, Wproj, bproj, W1, b1, W2, b2):
    raise NotImplementedError("write your pallas kernel here")


# speedup vs baseline: 93.5957x; 93.5957x over previous
import numpy as np
import jax
import jax.numpy as jnp
from jax.experimental import pallas as pl

QS = 64
C = 128
H = 4
RB = 8
NB = QS // RB
BATCH = 8

_MB = np.kron(np.eye(H, dtype=np.float32), np.ones((C // H, C // H), np.float32))


def _up(x, r, c, f):
    x = x.reshape(r, c, C)
    x = jnp.broadcast_to(x[:, :, None, :], (r, c, f, C)).reshape(r, c * f, C)
    x = jnp.broadcast_to(x[:, None, :, :], (r, f, c * f, C)).reshape(r * f, c * f, C)
    return x.reshape(r * f * c * f, C)


def _pool2(x, r, c):
    x = x.reshape(r, 2, 2 * c, C).sum(axis=1)
    x = x.reshape(r, c, 2, C).sum(axis=2)
    return x.reshape(r * c, C)


def _body(tgt, f0, f1, f2, f3, wq, bq, wkv0, bkv0, wkv1, bkv1, wkv2, bkv2,
          wkv3, bkv3, wproj, bproj, w1, b1, w2, b2, mb, out):
    x = tgt[0]
    mbm = mb[...]
    q = jnp.dot(x, wq[...], preferred_element_type=jnp.float32) + bq[...]

    def kv(feat, w, bvec):
        y = jnp.dot(feat, w[...], preferred_element_type=jnp.float32) + bvec[...]
        return y[:, :C], y[:, C:]

    def hd(a, b):
        return jnp.dot(a * b, mbm, preferred_element_type=jnp.float32)

    k1, v1 = kv(f1[0], wkv1, bkv1)
    acc = hd(q, k1) * v1

    k2, v2 = kv(f2[0], wkv2, bkv2)
    acc = acc + hd(q, _up(k2, RB // 2, QS // 2, 2)) * _up(v2, RB // 2, QS // 2, 2)

    k3, v3 = kv(f3[0], wkv3, bkv3)
    acc = acc + hd(q, _up(k3, RB // 4, QS // 4, 4)) * _up(v3, RB // 4, QS // 4, 4)

    k0, v0 = kv(f0[0], wkv0, bkv0)
    qu = _up(q, RB, QS, 2)
    acc = acc + _pool2(hd(qu, k0) * v0, RB, QS)

    y = x + jnp.dot(acc, wproj[...], preferred_element_type=jnp.float32) + bproj[...]
    z = jnp.maximum(jnp.dot(y, w1[...], preferred_element_type=jnp.float32) + b1[...], 0.0)
    out[0] = y + jnp.dot(z, w2[...], preferred_element_type=jnp.float32) + b2[...]


def kernel(tgt, feat0, feat1, feat2, feat3, Wq, bq, Wkv0, bkv0, Wkv1, bkv1,
           Wkv2, bkv2, Wkv3, bkv3, Wproj, bproj, W1, b1, W2, b2):
    Bsz, L, _ = tgt.shape
    full = lambda shape: pl.BlockSpec(shape, lambda b, r: (0,) * len(shape))
    blk = lambda n: pl.BlockSpec((1, n, C), lambda b, r: (b, r, 0))

    grid_spec = pl.GridSpec(
        grid=(Bsz, NB),
        in_specs=[
            blk(RB * QS),
            blk(RB * QS * 4),
            blk(RB * QS),
            blk(RB * QS // 4),
            blk(RB * QS // 16),
            full(Wq.shape),
            full((1, C)),
            full(Wkv0.shape), full((1, 2 * C)),
            full(Wkv1.shape), full((1, 2 * C)),
            full(Wkv2.shape), full((1, 2 * C)),
            full(Wkv3.shape), full((1, 2 * C)),
            full(Wproj.shape), full((1, C)),
            full(W1.shape), full((1, W1.shape[1])),
            full(W2.shape), full((1, C)),
            full(_MB.shape),
        ],
        out_specs=blk(RB * QS),
    )
    return pl.pallas_call(
        _body,
        grid_spec=grid_spec,
        out_shape=jax.ShapeDtypeStruct((Bsz, L, C), jnp.float32),
    )(tgt, feat0, feat1, feat2, feat3, Wq, bq.reshape(1, C),
      Wkv0, bkv0.reshape(1, -1), Wkv1, bkv1.reshape(1, -1),
      Wkv2, bkv2.reshape(1, -1), Wkv3, bkv3.reshape(1, -1),
      Wproj, bproj.reshape(1, -1), W1, b1.reshape(1, -1),
      W2, b2.reshape(1, -1), jnp.asarray(_MB))
